# baseline (speedup 1.0000x reference)
import functools
import math

import jax
import jax.numpy as jnp
from jax import lax
from jax.experimental import pallas as pl
from jax.experimental.pallas import tpu as pltpu
from jax.experimental.pallas import tpu_sc as plsc

EMB_D = 32
SCALE = math.sqrt(32.0)

NUM_CORES = 2
NUM_SUBCORES = 16
NW = NUM_CORES * NUM_SUBCORES

TOKENS_B = 4096
TOKENS_T = 200
SLAB = 128
NB = TOKENS_B // SLAB
NTR = TOKENS_T // 8

RB = 4
LOOKAHEAD = 3
NGROUP = TOKENS_T // RB


def _build():
    mesh = plsc.VectorSubcoreMesh(core_axis_name="c", subcore_axis_name="s")

    @functools.partial(
        pl.kernel,
        mesh=mesh,
        compiler_params=pltpu.CompilerParams(
            use_tc_tiling_on_sc=False, needs_layout_passes=False
        ),
        out_type=jax.ShapeDtypeStruct(
            (TOKENS_T, EMB_D // 8, NB, 8 * SLAB), jnp.float32
        ),
        scratch_types=[
            pltpu.VMEM((NTR, 8, SLAB), jnp.int32),
            pltpu.VMEM((RB, SLAB, EMB_D), jnp.float32),
            pltpu.VMEM((SLAB * 33,), jnp.float32),
            pltpu.VMEM((RB, EMB_D * SLAB), jnp.float32),
            pltpu.SemaphoreType.DMA,
            pltpu.SemaphoreType.DMA,
        ],
    )
    def emb_kernel(idx_hbm, table_hbm, out_hbm, idx_v, rows_v, pad_v, tile_v,
                   gsem, ssem):
        wid = lax.axis_index("s") * NUM_CORES + lax.axis_index("c")
        pltpu.sync_copy(idx_hbm.at[pl.ds(0, NTR), wid], idx_v)

        def fire_gather(t, buf):
            pltpu.async_copy(
                table_hbm.at[idx_v.at[t // 8, t % 8]], rows_v.at[buf], gsem
            )

        def wait_gather(t, buf):
            pltpu.make_async_copy(
                table_hbm.at[idx_v.at[t // 8, t % 8]], rows_v.at[buf], gsem
            ).wait()

        def fire_scatter(t, buf):
            for r in range(4):
                pltpu.async_copy(
                    tile_v.at[buf, pl.ds(r * 1024, 1024)],
                    out_hbm.at[t, r, wid], ssem,
                )

        def wait_scatter(t, buf):
            for r in range(4):
                pltpu.make_async_copy(
                    tile_v.at[buf, pl.ds(r * 1024, 1024)],
                    out_hbm.at[t, r, wid], ssem,
                ).wait()

        lane_iota = lax.iota(jnp.int32, 16)
        contig_ids = [lane_iota + r for r in range(8)]
        strided_ids = [lane_iota * 33 + r for r in range(8)]

        for z in range(LOOKAHEAD):
            fire_gather(z, z)

        def group_body(g, carry):
            for bb in range(RB):
                t = g * RB + bb
                buf_g = (bb + LOOKAHEAD) % RB

                if bb >= LOOKAHEAD:
                    wait_scatter(t - LOOKAHEAD, buf_g)
                else:
                    @pl.when(g > 0)
                    def _():
                        wait_scatter(t - LOOKAHEAD, buf_g)

                if bb < RB - LOOKAHEAD:
                    fire_gather(t + LOOKAHEAD, buf_g)
                else:
                    @pl.when(g < NGROUP - 1)
                    def _():
                        fire_gather(t + LOOKAHEAD, buf_g)

                wait_gather(t, bb)
                rv = rows_v.at[bb]
                tv = tile_v.at[bb]
                for j0 in range(0, SLAB, 8):
                    vals = [rv[j0 + dj, pl.ds(c0, 16)]
                            for dj in range(8) for c0 in (0, 16)]
                    scaled = [v * SCALE for v in vals]
                    for dj in range(8):
                        for h, c0 in enumerate((0, 16)):
                            a = (j0 + dj) * 33 + c0
                            pv = pad_v.at[pl.ds(a - a % 8, 24)]
                            plsc.store_scatter(pv, [contig_ids[a % 8]],
                                               scaled[2 * dj + h])
                def h2_loads(l0, d0):
                    vals = []
                    for dd in range(8):
                        d = d0 + dd
                        base = l0 * 528 + d
                        pv = pad_v.at[pl.ds(base - d % 8, 504)]
                        vals.append(plsc.load_gather(pv, [strided_ids[d % 8]]))
                    return vals

                def h2_stores(l0, d0, vals):
                    for dd in range(8):
                        d = d0 + dd
                        tv[pl.ds((d0 + dd) * SLAB + l0 * 16, 16)] = vals[dd]

                prev = None
                for l0 in range(8):
                    for d0 in range(0, EMB_D, 8):
                        cur = h2_loads(l0, d0)
                        if prev is not None:
                            h2_stores(*prev)
                        prev = (l0, d0, cur)
                h2_stores(*prev)
                fire_scatter(t, bb)
            return carry

        lax.fori_loop(0, NGROUP, group_body, 0)

        for z in range(LOOKAHEAD):
            t = TOKENS_T - LOOKAHEAD + z
            wait_scatter(t, t % RB)

    return emb_kernel


_emb_kernel = _build()


def kernel(tokens, embedding_weight):
    idx = (
        tokens.astype(jnp.int32)
        .T.reshape(NTR, 8, NB, SLAB)
        .transpose(0, 2, 1, 3)
    )
    out = _emb_kernel(idx, embedding_weight)
    return (
        out.reshape(TOKENS_T, EMB_D // 8, NB, 8, SLAB)
        .transpose(2, 4, 0, 1, 3)
        .reshape(TOKENS_B, TOKENS_T, EMB_D)
    )


# speedup vs baseline: 1.0153x; 1.0153x over previous
import functools
import math

import jax
import jax.numpy as jnp
from jax import lax
from jax.experimental import pallas as pl
from jax.experimental.pallas import tpu as pltpu
from jax.experimental.pallas import tpu_sc as plsc

EMB_D = 32
SCALE = math.sqrt(32.0)

NUM_CORES = 2
NUM_SUBCORES = 16
NW = NUM_CORES * NUM_SUBCORES

TOKENS_B = 4096
TOKENS_T = 200
SLAB = 128
NB = TOKENS_B // SLAB
NTR = TOKENS_T // 8

RB = 4
LOOKAHEAD = 3
NGROUP = TOKENS_T // RB


def _build():
    mesh = plsc.VectorSubcoreMesh(core_axis_name="c", subcore_axis_name="s")

    @functools.partial(
        pl.kernel,
        mesh=mesh,
        compiler_params=pltpu.CompilerParams(
            use_tc_tiling_on_sc=False, needs_layout_passes=False
        ),
        out_type=jax.ShapeDtypeStruct(
            (TOKENS_T, EMB_D // 8, NB, 8 * SLAB), jnp.float32
        ),
        scratch_types=[
            pltpu.VMEM((NTR, 8, SLAB), jnp.int32),
            pltpu.VMEM((RB, SLAB, SLAB), jnp.float32),
            pltpu.VMEM((SLAB * 33,), jnp.float32),
            pltpu.VMEM((RB, EMB_D * SLAB), jnp.float32),
            pltpu.SemaphoreType.DMA,
            pltpu.SemaphoreType.DMA,
        ],
    )
    def emb_kernel(idx_hbm, table_hbm, out_hbm, idx_v, rows_v, pad_v, tile_v,
                   gsem, ssem):
        wid = lax.axis_index("s") * NUM_CORES + lax.axis_index("c")
        pltpu.sync_copy(idx_hbm.at[pl.ds(0, NTR), wid], idx_v)

        def fire_gather(t, buf):
            pltpu.async_copy(
                table_hbm.at[idx_v.at[t // 8, t % 8]], rows_v.at[buf], gsem
            )

        def wait_gather(t, buf):
            pltpu.make_async_copy(
                table_hbm.at[idx_v.at[t // 8, t % 8]], rows_v.at[buf], gsem
            ).wait()

        def fire_scatter(t, buf):
            for r in range(4):
                pltpu.async_copy(
                    tile_v.at[buf, pl.ds(r * 1024, 1024)],
                    out_hbm.at[t, r, wid], ssem,
                )

        def wait_scatter(t, buf):
            for r in range(4):
                pltpu.make_async_copy(
                    tile_v.at[buf, pl.ds(r * 1024, 1024)],
                    out_hbm.at[t, r, wid], ssem,
                ).wait()

        lane_iota = lax.iota(jnp.int32, 16)
        contig_ids = [lane_iota + r for r in range(8)]
        strided_ids = [lane_iota * 33 + r for r in range(8)]

        for z in range(LOOKAHEAD):
            fire_gather(z, z)

        def group_body(g, carry):
            for bb in range(RB):
                t = g * RB + bb
                buf_g = (bb + LOOKAHEAD) % RB

                if bb >= LOOKAHEAD:
                    wait_scatter(t - LOOKAHEAD, buf_g)
                else:
                    @pl.when(g > 0)
                    def _():
                        wait_scatter(t - LOOKAHEAD, buf_g)

                if bb < RB - LOOKAHEAD:
                    fire_gather(t + LOOKAHEAD, buf_g)
                else:
                    @pl.when(g < NGROUP - 1)
                    def _():
                        fire_gather(t + LOOKAHEAD, buf_g)

                wait_gather(t, bb)
                rv = rows_v.at[bb]
                tv = tile_v.at[bb]
                for j0 in range(0, SLAB, 8):
                    vals = [rv[j0 + dj, pl.ds(c0, 16)]
                            for dj in range(8) for c0 in (0, 16)]
                    scaled = [v * SCALE for v in vals]
                    for dj in range(8):
                        for h, c0 in enumerate((0, 16)):
                            a = (j0 + dj) * 33 + c0
                            pv = pad_v.at[pl.ds(a - a % 8, 24)]
                            plsc.store_scatter(pv, [contig_ids[a % 8]],
                                               scaled[2 * dj + h])
                def h2_loads(l0, d0):
                    vals = []
                    for dd in range(8):
                        d = d0 + dd
                        base = l0 * 528 + d
                        pv = pad_v.at[pl.ds(base - d % 8, 504)]
                        vals.append(plsc.load_gather(pv, [strided_ids[d % 8]]))
                    return vals

                def h2_stores(l0, d0, vals):
                    for dd in range(8):
                        d = d0 + dd
                        tv[pl.ds((d0 + dd) * SLAB + l0 * 16, 16)] = vals[dd]

                prev = None
                for l0 in range(8):
                    for d0 in range(0, EMB_D, 8):
                        cur = h2_loads(l0, d0)
                        if prev is not None:
                            h2_stores(*prev)
                        prev = (l0, d0, cur)
                h2_stores(*prev)
                fire_scatter(t, bb)
            return carry

        lax.fori_loop(0, NGROUP, group_body, 0)

        for z in range(LOOKAHEAD):
            t = TOKENS_T - LOOKAHEAD + z
            wait_scatter(t, t % RB)

    return emb_kernel


_emb_kernel = _build()


def kernel(tokens, embedding_weight):
    idx = (
        tokens.astype(jnp.int32)
        .T.reshape(NTR, 8, NB, SLAB)
        .transpose(0, 2, 1, 3)
    )
    w128 = jnp.pad(embedding_weight, ((0, 0), (0, SLAB - EMB_D)))
    out = _emb_kernel(idx, w128)
    return (
        out.reshape(TOKENS_T, EMB_D // 8, NB, 8, SLAB)
        .transpose(2, 4, 0, 1, 3)
        .reshape(TOKENS_B, TOKENS_T, EMB_D)
    )
